# baseline (speedup 1.0000x reference)
import functools

import jax
import jax.numpy as jnp
from jax import lax
from jax.experimental import pallas as pl
from jax.experimental.pallas import tpu as pltpu
from jax.experimental.pallas import tpu_sc as plsc

L = 16
NC = 2
NS = 16
NW = NC * NS

E = 128
B = 4096
HALF = 64
D = 2 * HALF
E_PER_W = E // NW
BC = 128
CPE = B // BC
KHI = HALF // L


def _sc_body(ev_hbm, emb_hbm, w_hbm, b_hbm, out_hbm,
             ev_v, emb_v, w_v, b_v, buf, sem0, sem1, sem2, sem3):
    wid = lax.axis_index("s") * NC + lax.axis_index("c")
    e0 = wid * E_PER_W
    pltpu.sync_copy(w_hbm, w_v.at[pl.ds(0, HALF - 1)])
    pltpu.sync_copy(b_hbm, b_v.at[pl.ds(0, HALF - 1)])
    pltpu.sync_copy(emb_hbm.at[pl.ds(e0, E_PER_W)], emb_v)
    pltpu.sync_copy(ev_hbm.at[pl.ds(e0, E_PER_W)], ev_v)
    sems = (sem0, sem1, sem2, sem3)

    lane = lax.iota(jnp.int32, L)
    is_last = lane == (L - 1)
    m_hi = [w_v[pl.ds(k * L, L)] for k in range(KHI)]
    m_hi[KHI - 1] = jnp.where(is_last, 1.0, m_hi[KHI - 1])
    c_hi = [b_v[pl.ds(k * L, L)] for k in range(KHI)]
    c_hi[KHI - 1] = jnp.where(is_last, 0.0, c_hi[KHI - 1])

    for d in range(E_PER_W):
        c_lo = [emb_v[d, pl.ds(k * L, L)] for k in range(KHI)]

        def prebuild(i, pcarry, _d=d, _c_lo=c_lo):
            for k in range(KHI):
                buf[_d, i, pl.ds(k * L, L)] = _c_lo[k]
            return pcarry

        lax.fori_loop(0, BC, prebuild, 0, unroll=4)

    def quad_body(q, carry):
        b0 = q * BC
        for d in range(E_PER_W):

            @pl.when(q > 0)
            def _wait():
                pltpu.make_async_copy(
                    buf.at[d], out_hbm.at[e0 + d, pl.ds(b0, BC)], sems[d]
                ).wait()

            def fill(ii, fcarry, _d=d):
                ev16 = ev_v[_d, pl.ds(b0 + ii * L, L)]
                for l in range(L):
                    ev = jnp.broadcast_to(ev16[l], (L,))
                    row = ii * L + l
                    for k in range(KHI):
                        buf[_d, row, pl.ds(HALF + k * L, L)] = (
                            ev * m_hi[k] + c_hi[k]
                        )
                return fcarry

            lax.fori_loop(0, BC // L, fill, 0, unroll=2)
            pltpu.async_copy(
                buf.at[d], out_hbm.at[e0 + d, pl.ds(b0, BC)], sems[d]
            )
        return carry

    lax.fori_loop(0, CPE, quad_body, 0)
    for d in range(E_PER_W):
        pltpu.make_async_copy(
            buf.at[d], out_hbm.at[e0, pl.ds(0, BC)], sems[d]
        ).wait()


def kernel(evals, emb_table, W, b):
    run = functools.partial(
        pl.kernel,
        out_type=jax.ShapeDtypeStruct((E, B, D), jnp.float32),
        mesh=plsc.VectorSubcoreMesh(core_axis_name="c", subcore_axis_name="s"),
        scratch_types=[
            pltpu.VMEM((E_PER_W, B), jnp.float32),
            pltpu.VMEM((E_PER_W, HALF), jnp.float32),
            pltpu.VMEM((HALF,), jnp.float32),
            pltpu.VMEM((HALF,), jnp.float32),
            pltpu.VMEM((E_PER_W, BC, D), jnp.float32),
            pltpu.SemaphoreType.DMA,
            pltpu.SemaphoreType.DMA,
            pltpu.SemaphoreType.DMA,
            pltpu.SemaphoreType.DMA,
        ],
    )(_sc_body)
    return run(evals, emb_table, W[:, 0], b)


# speedup vs baseline: 1.0345x; 1.0345x over previous
import functools

import jax
import jax.numpy as jnp
from jax import lax
from jax.experimental import pallas as pl
from jax.experimental.pallas import tpu as pltpu
from jax.experimental.pallas import tpu_sc as plsc

L = 16
NC = 2
NS = 16
NW = NC * NS

E = 128
B = 4096
HALF = 64
D = 2 * HALF
E_PER_W = E // NW
BC = 256
CPE = B // BC
NCHUNK = E_PER_W * CPE
KHI = HALF // L


def _sc_body(ev_hbm, emb_hbm, w_hbm, b_hbm, out_hbm,
             ev_v, emb_v, w_v, b_v, buf, sem0, sem1):
    wid = lax.axis_index("s") * NC + lax.axis_index("c")
    e0 = wid * E_PER_W
    pltpu.sync_copy(w_hbm, w_v.at[pl.ds(0, HALF - 1)])
    pltpu.sync_copy(b_hbm, b_v.at[pl.ds(0, HALF - 1)])
    pltpu.sync_copy(emb_hbm.at[pl.ds(e0, E_PER_W)], emb_v)
    pltpu.sync_copy(ev_hbm.at[pl.ds(e0, E_PER_W)], ev_v)
    sems = (sem0, sem1)

    lane = lax.iota(jnp.int32, L)
    is_last = lane == (L - 1)
    m_hi = [w_v[pl.ds(k * L, L)] for k in range(KHI)]
    m_hi[KHI - 1] = jnp.where(is_last, 1.0, m_hi[KHI - 1])
    c_hi = [b_v[pl.ds(k * L, L)] for k in range(KHI)]
    c_hi[KHI - 1] = jnp.where(is_last, 0.0, c_hi[KHI - 1])

    def pair_body(p, carry):
        for d in range(2):
            chunk = 2 * p + d
            ei = 2 * (chunk // (2 * CPE)) + d
            bi = (chunk // 2) % CPE
            b0 = bi * BC

            @pl.when(p > 0)
            def _wait():
                pltpu.make_async_copy(
                    buf.at[d], out_hbm.at[e0 + ei, pl.ds(b0, BC)], sems[d]
                ).wait()

            @pl.when(bi == 0)
            def _rebuild_const_half():
                c_lo = [emb_v[ei, pl.ds(k * L, L)] for k in range(KHI)]

                def prebuild(i, pcarry, _d=d):
                    for k in range(KHI):
                        buf[_d, i, pl.ds(k * L, L)] = c_lo[k]
                    return pcarry

                lax.fori_loop(0, BC, prebuild, 0, unroll=4)

            def fill(ii, fcarry, _d=d):
                ev16 = ev_v[ei, pl.ds(b0 + ii * L, L)]
                for l in range(L):
                    ev = jnp.broadcast_to(ev16[l], (L,))
                    row = ii * L + l
                    for k in range(KHI):
                        buf[_d, row, pl.ds(HALF + k * L, L)] = (
                            ev * m_hi[k] + c_hi[k]
                        )
                return fcarry

            lax.fori_loop(0, BC // L, fill, 0, unroll=4)
            pltpu.async_copy(
                buf.at[d], out_hbm.at[e0 + ei, pl.ds(b0, BC)], sems[d]
            )
        return carry

    lax.fori_loop(0, NCHUNK // 2, pair_body, 0)
    for d in range(2):
        pltpu.make_async_copy(
            buf.at[d], out_hbm.at[e0, pl.ds(0, BC)], sems[d]
        ).wait()


def kernel(evals, emb_table, W, b):
    run = functools.partial(
        pl.kernel,
        out_type=jax.ShapeDtypeStruct((E, B, D), jnp.float32),
        mesh=plsc.VectorSubcoreMesh(core_axis_name="c", subcore_axis_name="s"),
        scratch_types=[
            pltpu.VMEM((E_PER_W, B), jnp.float32),
            pltpu.VMEM((E_PER_W, HALF), jnp.float32),
            pltpu.VMEM((HALF,), jnp.float32),
            pltpu.VMEM((HALF,), jnp.float32),
            pltpu.VMEM((2, BC, D), jnp.float32),
            pltpu.SemaphoreType.DMA,
            pltpu.SemaphoreType.DMA,
        ],
    )(_sc_body)
    return run(evals, emb_table, W[:, 0], b)


# speedup vs baseline: 1.0431x; 1.0084x over previous
import functools

import jax
import jax.numpy as jnp
from jax import lax
from jax.experimental import pallas as pl
from jax.experimental.pallas import tpu as pltpu
from jax.experimental.pallas import tpu_sc as plsc

L = 16
NC = 2
NS = 16
NW = NC * NS

E = 128
B = 4096
HALF = 64
D = 2 * HALF
E_PER_W = E // NW
BC = 256
CPE = B // BC
NCHUNK = E_PER_W * CPE
KHI = HALF // L


def _sc_body(ev_hbm, emb_hbm, w_hbm, b_hbm, out_hbm,
             ev_v, emb_v, w_v, b_v, buf, sem0, sem1):
    wid = lax.axis_index("s") * NC + lax.axis_index("c")
    e0 = wid * E_PER_W
    pltpu.sync_copy(w_hbm, w_v.at[pl.ds(0, HALF - 1)])
    pltpu.sync_copy(b_hbm, b_v.at[pl.ds(0, HALF - 1)])
    pltpu.sync_copy(emb_hbm.at[pl.ds(e0, E_PER_W)], emb_v)
    pltpu.sync_copy(ev_hbm.at[pl.ds(e0, E_PER_W)], ev_v)
    sems = (sem0, sem1)

    lane = lax.iota(jnp.int32, L)
    is_last = lane == (L - 1)
    m_hi = [w_v[pl.ds(k * L, L)] for k in range(KHI)]
    m_hi[KHI - 1] = jnp.where(is_last, 1.0, m_hi[KHI - 1])
    c_hi = [b_v[pl.ds(k * L, L)] for k in range(KHI)]
    c_hi[KHI - 1] = jnp.where(is_last, 0.0, c_hi[KHI - 1])

    def pair_body(p, carry):
        for d in range(2):
            chunk = 2 * p + d
            ei = 2 * (chunk // (2 * CPE)) + d
            bi = (chunk // 2) % CPE
            b0 = bi * BC

            @pl.when(p > 0)
            def _wait():
                pltpu.make_async_copy(
                    buf.at[d], out_hbm.at[e0 + ei, pl.ds(b0, BC)], sems[d]
                ).wait()

            @pl.when(bi == 0)
            def _rebuild_const_half():
                c_lo = [emb_v[ei, pl.ds(k * L, L)] for k in range(KHI)]

                def prebuild(i, pcarry, _d=d):
                    for k in range(KHI):
                        buf[_d, i, pl.ds(k * L, L)] = c_lo[k]
                    return pcarry

                lax.fori_loop(0, BC, prebuild, 0, unroll=4)

            def fill(ii, fcarry, _d=d):
                ev16 = ev_v[ei, pl.ds(b0 + ii * L, L)]
                for l in range(L):
                    ev = jnp.broadcast_to(ev16[l], (L,))
                    row = ii * L + l
                    for k in range(KHI):
                        buf[_d, row, pl.ds(HALF + k * L, L)] = (
                            ev * m_hi[k] + c_hi[k]
                        )
                return fcarry

            lax.fori_loop(0, BC // L, fill, 0, unroll=2)
            pltpu.async_copy(
                buf.at[d], out_hbm.at[e0 + ei, pl.ds(b0, BC)], sems[d]
            )
        return carry

    lax.fori_loop(0, NCHUNK // 2, pair_body, 0)
    for d in range(2):
        pltpu.make_async_copy(
            buf.at[d], out_hbm.at[e0, pl.ds(0, BC)], sems[d]
        ).wait()


def kernel(evals, emb_table, W, b):
    run = functools.partial(
        pl.kernel,
        out_type=jax.ShapeDtypeStruct((E, B, D), jnp.float32),
        mesh=plsc.VectorSubcoreMesh(core_axis_name="c", subcore_axis_name="s"),
        scratch_types=[
            pltpu.VMEM((E_PER_W, B), jnp.float32),
            pltpu.VMEM((E_PER_W, HALF), jnp.float32),
            pltpu.VMEM((HALF,), jnp.float32),
            pltpu.VMEM((HALF,), jnp.float32),
            pltpu.VMEM((2, BC, D), jnp.float32),
            pltpu.SemaphoreType.DMA,
            pltpu.SemaphoreType.DMA,
        ],
    )(_sc_body)
    return run(evals, emb_table, W[:, 0], b)
